# baseline (speedup 1.0000x reference)
import jax
import jax.numpy as jnp
from jax import lax
from jax.experimental import pallas as pl
from jax.experimental.pallas import tpu as pltpu
from jax.experimental.pallas import tpu_sc as plsc

_BINS = 30
_MMT = 0.6
_BATCH = 16384
_CLASS = 64

_NC = 2
_NS = 16
_NW = _NC * _NS
_L = 16

_ROWS_PER_TILE = _BATCH // _NW
_PAD = 128


_CH = 128
_NCHUNK = _ROWS_PER_TILE // _CH


def _hist_body(pred_hbm, tgt_hbm, tab_hbm, p0, p1, t0, t1, cnt_v,
               bsum_v, sp0, sp1, st0, st1):
    wid = lax.axis_index("s") * _NC + lax.axis_index("c")
    rbase = wid * _ROWS_PER_TILE
    pbufs, tbufs = [p0, p1], [t0, t1]
    psems, tsems = [sp0, sp1], [st0, st1]

    def start(ch):
        return (
            pltpu.async_copy(pred_hbm.at[pl.ds(rbase + ch * _CH, _CH), :],
                             pbufs[ch % 2], psems[ch % 2]),
            pltpu.async_copy(tgt_hbm.at[pl.ds(rbase + ch * _CH, _CH), :],
                             tbufs[ch % 2], tsems[ch % 2]),
        )

    cps = {0: start(0), 1: start(1)}

    zeros = jnp.zeros((_L,), jnp.float32)

    def zero_body(r, carry):
        for j in range(2):
            cnt_v[r, pl.ds(j * _L, _L)] = zeros
            bsum_v[r, pl.ds(j * _L, _L)] = zeros
        return carry

    lax.fori_loop(0, _CLASS, zero_body, 0, unroll=4)

    lane = lax.iota(jnp.int32, _L)
    ones = jnp.ones((_L,), jnp.float32)
    cls_idx = [k * _L + lane for k in range(_CLASS // _L)]

    c = (2.211703e-05, 0.99901044, -0.48915684, 0.28330433, -0.13011941,
         0.030102625)

    _RPI = 4
    _KV = _RPI * (_CLASS // _L)

    def make_body(pv, tv):
        def row_body(it, carry):
            r0 = it * _RPI
            ps = [pv[r0 + k // 4, pl.ds((k % 4) * _L, _L)] for k in range(_KV)]
            ts = [tv[r0 + k // 4, pl.ds((k % 4) * _L, _L)] for k in range(_KV)]
            qs = [jnp.where(t > 0, -p, p) for p, t in zip(ps, ts)]
            us = [jnp.exp(-jnp.abs(p)) for p in ps]
            for k in range(_KV):
                q, u = qs[k], us[k]
                l1p = c[5]
                for j in range(4, -1, -1):
                    l1p = l1p * u + c[j]
                bce = jnp.maximum(q, 0.0) + l1p
                r1 = 1.0 / (1.0 + u)
                g = jnp.where(q >= 0.0, r1, u * r1)
                b = jnp.minimum((g * float(_BINS)).astype(jnp.int32), _BINS - 1)
                ci = cls_idx[k % (_CLASS // _L)]
                plsc.addupdate_scatter(cnt_v, [ci, b], ones)
                plsc.addupdate_scatter(bsum_v, [ci, b], bce)
            return carry
        return row_body

    for ch in range(_NCHUNK):
        cp_p, cp_t = cps[ch]
        cp_p.wait()
        cp_t.wait()
        lax.fori_loop(0, _CH // _RPI, make_body(pbufs[ch % 2], tbufs[ch % 2]),
                      0, unroll=2)
        if ch + 2 < _NCHUNK:
            cps[ch + 2] = start(ch + 2)

    pltpu.sync_copy(cnt_v, tab_hbm.at[0, wid])
    pltpu.sync_copy(bsum_v, tab_hbm.at[1, wid])


_hist = pl.kernel(
    _hist_body,
    out_type=jax.ShapeDtypeStruct((2, _NW, _CLASS, _PAD), jnp.float32),
    mesh=plsc.VectorSubcoreMesh(core_axis_name="c", subcore_axis_name="s"),
    compiler_params=pltpu.CompilerParams(needs_layout_passes=False),
    scratch_types=[
        pltpu.VMEM((_CH, _CLASS), jnp.float32),
        pltpu.VMEM((_CH, _CLASS), jnp.float32),
        pltpu.VMEM((_CH, _CLASS), jnp.int32),
        pltpu.VMEM((_CH, _CLASS), jnp.int32),
        pltpu.VMEM((_CLASS, _PAD), jnp.float32),
        pltpu.VMEM((_CLASS, _PAD), jnp.float32),
        pltpu.SemaphoreType.DMA,
        pltpu.SemaphoreType.DMA,
        pltpu.SemaphoreType.DMA,
        pltpu.SemaphoreType.DMA,
    ],
)


def _epi_body(tab_ref, acc_ref, out_ref):
    tab = tab_ref[...]
    cnt = jnp.sum(tab[0], axis=0)[:, :_BINS]
    bsum = jnp.sum(tab[1], axis=0)[:, :_BINS]
    acc = acc_ref[...]
    ne = cnt > 0.0
    accn = jnp.where(ne, _MMT * acc + (1.0 - _MMT) * cnt, acc)
    contrib = jnp.where(ne, bsum / jnp.where(ne, accn, 1.0), 0.0)
    n = jnp.sum(ne.astype(jnp.float32), axis=1)
    csum = jnp.sum(contrib, axis=1)
    n = jnp.where(n > 0.0, n, 1.0)
    loss = jnp.sum(csum / n) * (1.0 / _CLASS)
    out_ref[...] = loss[None, None]


_epilogue = pl.pallas_call(
    _epi_body,
    out_shape=jax.ShapeDtypeStruct((1, 1), jnp.float32),
)


def kernel(pred, target, acc_sum):
    tab = _hist(pred, target)
    loss2d = _epilogue(tab, acc_sum)
    return loss2d[0, 0]
